# baseline (speedup 1.0000x reference)
import functools

import jax
import jax.numpy as jnp
from jax import lax
from jax.experimental import pallas as pl
from jax.experimental.pallas import tpu as pltpu
from jax.experimental.pallas import tpu_sc as plsc

B, D, K = 16384, 1024, 128
NC, NS, L = 2, 16, 16
NW = NC * NS
ROWS_W = B // NW
R = 16
NCH = ROWS_W // R
NBUF = 2


def _row_compute(cb, db, idx_ref, r):
    rsplat = jnp.full((L,), r, jnp.int32)
    cs, ds = [], []
    for j in range(K // L):
        ij = idx_ref[pl.ds(j * L, L)]
        cs.append(plsc.load_gather(cb, [rsplat, ij]))
        ds.append(plsc.load_gather(db, [rsplat, ij]))
    tgt0 = lax.iota(jnp.int32, L)
    for j in range(K // L):
        lo = jnp.maximum(cs[j] - ds[j], 0.0)
        hi = jnp.maximum(cs[j] + ds[j], 0.0)
        tj = tgt0 + (j * L)
        plsc.store_scatter(cb, [rsplat, tj], (lo + hi) * 0.5)
        plsc.store_scatter(db, [rsplat, tj], (hi - lo) * 0.5)


def _sc_body(c_hbm, d_hbm, idx_hbm, out_hbm, idx_v, cbuf, dbuf, in_sem, out_sem):
    wid = lax.axis_index("s") * NC + lax.axis_index("c")
    base = wid * ROWS_W
    pltpu.sync_copy(idx_hbm, idx_v)

    def in_copies(g, b):
        row0 = base + g * R
        return (
            pltpu.make_async_copy(c_hbm.at[pl.ds(row0, R)], cbuf.at[b], in_sem),
            pltpu.make_async_copy(d_hbm.at[pl.ds(row0, R)], dbuf.at[b], in_sem),
        )

    def out_copies(g, b):
        row0 = base + g * R
        return (
            pltpu.make_async_copy(cbuf.at[b], out_hbm.at[0].at[pl.ds(row0, R)], out_sem),
            pltpu.make_async_copy(dbuf.at[b], out_hbm.at[1].at[pl.ds(row0, R)], out_sem),
        )

    for copy in in_copies(0, 0):
        copy.start()
    for copy in in_copies(1, 1):
        copy.start()

    def chunk(g, b):
        for copy in in_copies(g, b):
            copy.wait()

        def row_body(r, carry):
            _row_compute(cbuf.at[b], dbuf.at[b], idx_v, r)
            return carry

        lax.fori_loop(0, R, row_body, 0)
        for copy in out_copies(g, b):
            copy.start()
        for copy in out_copies(g, b):
            copy.wait()

        @pl.when(g + NBUF < NCH)
        def _():
            for copy in in_copies(g + NBUF, b):
                copy.start()

    def outer(g0, carry):
        for bb in range(NBUF):
            chunk(g0 + bb, bb)
        return carry

    lax.fori_loop(0, NCH // NBUF, lambda i, cr: outer(i * NBUF, cr), 0)


@jax.jit
def kernel(c, delta, arg_idx, target_idx):
    del target_idx
    sc_fn = functools.partial(
        pl.kernel,
        out_type=jax.ShapeDtypeStruct((2, B, D), jnp.float32),
        mesh=plsc.VectorSubcoreMesh(core_axis_name="c", subcore_axis_name="s"),
        scratch_types=[
            pltpu.VMEM((K,), jnp.int32),
            pltpu.VMEM((NBUF, R, D), jnp.float32),
            pltpu.VMEM((NBUF, R, D), jnp.float32),
            pltpu.SemaphoreType.DMA,
            pltpu.SemaphoreType.DMA,
        ],
        compiler_params=pltpu.CompilerParams(use_tc_tiling_on_sc=False, needs_layout_passes=False),
    )(_sc_body)
    return sc_fn(c, delta, arg_idx)


# speedup vs baseline: 3.0942x; 3.0942x over previous
import functools

import jax
import jax.numpy as jnp
from jax import lax
from jax.experimental import pallas as pl
from jax.experimental.pallas import tpu as pltpu
from jax.experimental.pallas import tpu_sc as plsc

B, D, K = 16384, 1024, 128
NC, NS, L = 2, 16, 16
NW = NC * NS
ROWS_W = B // NW
R = 16
NCH = ROWS_W // R
NBUF = 2


def _row_compute(cb, db, idx_ref, r):
    rsplat = jnp.full((L,), r, jnp.int32)
    cs, ds = [], []
    for j in range(K // L):
        ij = idx_ref[pl.ds(j * L, L)]
        cs.append(plsc.load_gather(cb, [rsplat, ij]))
        ds.append(plsc.load_gather(db, [rsplat, ij]))
    tgt0 = lax.iota(jnp.int32, L)
    for j in range(K // L):
        lo = jnp.maximum(cs[j] - ds[j], 0.0)
        hi = jnp.maximum(cs[j] + ds[j], 0.0)
        tj = tgt0 + (j * L)
        plsc.store_scatter(cb, [rsplat, tj], (lo + hi) * 0.5)
        plsc.store_scatter(db, [rsplat, tj], (hi - lo) * 0.5)


def _sc_body(c_hbm, d_hbm, idx_hbm, out_hbm, idx_v, cbuf, dbuf, in_sem, out_sem):
    wid = lax.axis_index("s") * NC + lax.axis_index("c")
    base = wid * ROWS_W
    pltpu.sync_copy(idx_hbm, idx_v)

    def in_copies(g, b):
        row0 = base + g * R
        return (
            pltpu.make_async_copy(c_hbm.at[pl.ds(row0, R)], cbuf.at[b], in_sem),
            pltpu.make_async_copy(d_hbm.at[pl.ds(row0, R)], dbuf.at[b], in_sem),
        )

    def out_copies(g, b):
        row0 = base + g * R
        return (
            pltpu.make_async_copy(cbuf.at[b], out_hbm.at[0].at[pl.ds(row0, R)], out_sem),
            pltpu.make_async_copy(dbuf.at[b], out_hbm.at[1].at[pl.ds(row0, R)], out_sem),
        )

    for copy in in_copies(0, 0):
        copy.start()
    for copy in in_copies(1, 1):
        copy.start()

    def chunk(g, b):
        for copy in in_copies(g, b):
            copy.wait()

        def row_body(r, carry):
            _row_compute(cbuf.at[b], dbuf.at[b], idx_v, r)
            return carry

        lax.fori_loop(0, R, row_body, 0)
        for copy in out_copies(g, b):
            copy.start()
        for copy in out_copies(g, b):
            copy.wait()

        @pl.when(g + NBUF < NCH)
        def _():
            for copy in in_copies(g + NBUF, b):
                copy.start()

    def outer(g0, carry):
        for bb in range(NBUF):
            chunk(g0 + bb, bb)
        return carry

    lax.fori_loop(0, NCH // NBUF, lambda i, cr: outer(i * NBUF, cr), 0)


@jax.jit
def kernel(c, delta, arg_idx, target_idx):
    del target_idx
    sc_fn = functools.partial(
        pl.kernel,
        out_type=jax.ShapeDtypeStruct((2, B, D), jnp.float32),
        mesh=plsc.VectorSubcoreMesh(core_axis_name="c", subcore_axis_name="s"),
        scratch_types=[
            pltpu.VMEM((K,), jnp.int32),
            pltpu.VMEM((NBUF, R, D), jnp.float32),
            pltpu.VMEM((NBUF, R, D), jnp.float32),
            pltpu.SemaphoreType.DMA,
            pltpu.SemaphoreType.DMA,
        ],
        compiler_params=pltpu.CompilerParams(use_tc_tiling_on_sc=True, needs_layout_passes=False),
    )(_sc_body)
    return sc_fn(c, delta, arg_idx)


# speedup vs baseline: 3.1218x; 1.0089x over previous
import functools

import jax
import jax.numpy as jnp
from jax import lax
from jax.experimental import pallas as pl
from jax.experimental.pallas import tpu as pltpu
from jax.experimental.pallas import tpu_sc as plsc

B, D, K = 16384, 1024, 128
NC, NS, L = 2, 16, 16
NW = NC * NS
ROWS_W = B // NW
R = 16
NCH = ROWS_W // R
NBUF = 3


def _row_compute(cb, db, idx_ref, r):
    rsplat = jnp.full((L,), r, jnp.int32)
    cs, ds = [], []
    for j in range(K // L):
        ij = idx_ref[pl.ds(j * L, L)]
        cs.append(plsc.load_gather(cb, [rsplat, ij]))
        ds.append(plsc.load_gather(db, [rsplat, ij]))
    tgt0 = lax.iota(jnp.int32, L)
    for j in range(K // L):
        lo = jnp.maximum(cs[j] - ds[j], 0.0)
        hi = jnp.maximum(cs[j] + ds[j], 0.0)
        tj = tgt0 + (j * L)
        plsc.store_scatter(cb, [rsplat, tj], (lo + hi) * 0.5)
        plsc.store_scatter(db, [rsplat, tj], (hi - lo) * 0.5)


def _sc_body(c_hbm, d_hbm, idx_hbm, out_hbm, idx_v, cbuf, dbuf, in_sem, out_sem):
    wid = lax.axis_index("s") * NC + lax.axis_index("c")
    base = wid * ROWS_W
    pltpu.sync_copy(idx_hbm, idx_v)

    def in_copies(g, b):
        row0 = base + g * R
        return (
            pltpu.make_async_copy(c_hbm.at[pl.ds(row0, R)], cbuf.at[b], in_sem),
            pltpu.make_async_copy(d_hbm.at[pl.ds(row0, R)], dbuf.at[b], in_sem),
        )

    def out_copies(g, b):
        row0 = base + g * R
        return (
            pltpu.make_async_copy(cbuf.at[b], out_hbm.at[0].at[pl.ds(row0, R)], out_sem),
            pltpu.make_async_copy(dbuf.at[b], out_hbm.at[1].at[pl.ds(row0, R)], out_sem),
        )

    for bb in range(NBUF):
        for copy in in_copies(bb, bb):
            copy.start()

    def chunk(g, b, b_next):
        @pl.when(g < NCH)
        def _():
            @pl.when((g + 1 >= NBUF) & (g + 1 < NCH))
            def _():
                for copy in out_copies(g + 1 - NBUF, b_next):
                    copy.wait()
                for copy in in_copies(g + 1, b_next):
                    copy.start()

            for copy in in_copies(g, b):
                copy.wait()

            def row_body(r, carry):
                _row_compute(cbuf.at[b], dbuf.at[b], idx_v, r)
                return carry

            lax.fori_loop(0, R, row_body, 0)
            for copy in out_copies(g, b):
                copy.start()

    def outer(g0, carry):
        for bb in range(NBUF):
            chunk(g0 + bb, bb, (bb + 1) % NBUF)
        return carry

    n_outer = (NCH + NBUF - 1) // NBUF
    lax.fori_loop(0, n_outer, lambda i, cr: outer(i * NBUF, cr), 0)
    for g in range(NCH - NBUF, NCH):
        for copy in out_copies(g, g % NBUF):
            copy.wait()


@jax.jit
def kernel(c, delta, arg_idx, target_idx):
    del target_idx
    sc_fn = functools.partial(
        pl.kernel,
        out_type=jax.ShapeDtypeStruct((2, B, D), jnp.float32),
        mesh=plsc.VectorSubcoreMesh(core_axis_name="c", subcore_axis_name="s"),
        scratch_types=[
            pltpu.VMEM((K,), jnp.int32),
            pltpu.VMEM((NBUF, R, D), jnp.float32),
            pltpu.VMEM((NBUF, R, D), jnp.float32),
            pltpu.SemaphoreType.DMA,
            pltpu.SemaphoreType.DMA,
        ],
        compiler_params=pltpu.CompilerParams(use_tc_tiling_on_sc=True, needs_layout_passes=False),
    )(_sc_body)
    return sc_fn(c, delta, arg_idx)
